# baseline (speedup 1.0000x reference)
import functools

import jax
import jax.numpy as jnp
from jax import lax
from jax.experimental import pallas as pl
from jax.experimental.pallas import tpu as pltpu
from jax.experimental.pallas import tpu_sc as plsc

NUM_NODES = 100000
FEAT = 128
NUM_GRAPHS = 64

NUM_CORES = 2
NUM_SUBCORES = 16
NUM_WORKERS = NUM_CORES * NUM_SUBCORES
CHUNK = 3200
PAD_N = NUM_WORKERS * CHUNK
PAD_VALUE = NUM_GRAPHS
HIST = 128

BLOCK_ROWS = 10000
GRID = NUM_NODES // BLOCK_ROWS


def _sc_hist_body(batch_ref, out_ref, idx_v, ones_v, zeros_v, shared, sem):
    cid = lax.axis_index("c")
    sid = lax.axis_index("s")
    wid = sid * NUM_CORES + cid

    load_idx = pltpu.async_copy(batch_ref.at[pl.ds(wid * CHUNK, CHUNK)], idx_v, sem)

    ones16 = jnp.ones((16,), jnp.float32)

    def _fill(j, carry):
        ones_v[pl.ds(j * 16, 16)] = ones16
        return carry

    lax.fori_loop(0, CHUNK // 16, _fill, 0)
    for j in range(HIST // 16):
        zeros_v[pl.ds(j * 16, 16)] = jnp.zeros((16,), jnp.float32)

    @pl.when(sid == 0)
    def _():
        pltpu.sync_copy(zeros_v, shared)

    load_idx.wait()
    plsc.subcore_barrier()

    pltpu.sync_copy(ones_v, shared.at[idx_v], add=True)

    plsc.subcore_barrier()

    @pl.when(sid == 0)
    def _():
        pltpu.sync_copy(shared, out_ref.at[cid])


@functools.cache
def _sc_hist():
    return pl.kernel(
        _sc_hist_body,
        out_type=jax.ShapeDtypeStruct((NUM_CORES, HIST), jnp.float32),
        mesh=plsc.VectorSubcoreMesh(core_axis_name="c", subcore_axis_name="s"),
        scratch_types=[
            pltpu.VMEM((CHUNK,), jnp.int32),
            pltpu.VMEM((CHUNK,), jnp.float32),
            pltpu.VMEM((HIST,), jnp.float32),
            pltpu.VMEM_SHARED((HIST,), jnp.float32),
            pltpu.SemaphoreType.DMA,
        ],
    )


def _tc_scale_body(deg_ref, x_ref, o_ref):
    i = pl.program_id(0)
    deg = deg_ref[0:1, :] + deg_ref[1:2, :]
    inv = jnp.where(deg > 0.0, lax.rsqrt(deg), 0.0)
    row_ix = lax.broadcasted_iota(jnp.int32, (HIST, HIST), 0)
    col_ix = lax.broadcasted_iota(jnp.int32, (HIST, HIST), 1)
    tri = (row_ix < col_ix).astype(jnp.float32)
    s_excl = jnp.dot(
        deg, tri, precision=lax.Precision.HIGHEST, preferred_element_type=jnp.float32
    )
    s_next = s_excl + deg
    gr = (lax.broadcasted_iota(jnp.int32, (BLOCK_ROWS, 1), 0) + i * BLOCK_ROWS).astype(
        jnp.float32
    )
    member = jnp.logical_and(gr >= s_excl, gr < s_next).astype(jnp.float32)
    scale = jnp.dot(
        member,
        inv.reshape(HIST, 1),
        precision=lax.Precision.HIGHEST,
        preferred_element_type=jnp.float32,
    )
    o_ref[...] = x_ref[...] * scale


def kernel(x, batch):
    batch = batch.astype(jnp.int32)
    pad = jnp.full((PAD_N - NUM_NODES,), PAD_VALUE, jnp.int32)
    batch1d = jnp.concatenate([batch, pad])
    deg2 = _sc_hist()(batch1d)

    return pl.pallas_call(
        _tc_scale_body,
        grid=(GRID,),
        in_specs=[
            pl.BlockSpec((NUM_CORES, HIST), lambda i: (0, 0)),
            pl.BlockSpec((BLOCK_ROWS, FEAT), lambda i: (i, 0)),
        ],
        out_specs=pl.BlockSpec((BLOCK_ROWS, FEAT), lambda i: (i, 0)),
        out_shape=jax.ShapeDtypeStruct((NUM_NODES, FEAT), jnp.float32),
        compiler_params=pltpu.CompilerParams(
            dimension_semantics=("arbitrary",),
        ),
    )(deg2, x)


# speedup vs baseline: 1.3355x; 1.3355x over previous
import functools

import jax
import jax.numpy as jnp
from jax import lax
from jax.experimental import pallas as pl
from jax.experimental.pallas import tpu as pltpu
from jax.experimental.pallas import tpu_sc as plsc

NUM_NODES = 100000
FEAT = 128
NUM_GRAPHS = 64

NUM_CORES = 2
NUM_SUBCORES = 16
NUM_WORKERS = NUM_CORES * NUM_SUBCORES
CHUNK = 3200
PAD_N = NUM_WORKERS * CHUNK
PAD_VALUE = NUM_GRAPHS
HIST = 128

BLOCK_ROWS = 10000
GRID = NUM_NODES // BLOCK_ROWS


def _sc_hist_body(batch_ref, out_ref, idx_v, ones_v, zeros_v, shared, sem):
    cid = lax.axis_index("c")
    sid = lax.axis_index("s")
    wid = sid * NUM_CORES + cid

    load_idx = pltpu.async_copy(batch_ref.at[pl.ds(wid * CHUNK, CHUNK)], idx_v, sem)

    ones16 = jnp.ones((16,), jnp.float32)

    def _fill(j, carry):
        ones_v[pl.ds(j * 16, 16)] = ones16
        return carry

    lax.fori_loop(0, CHUNK // 16, _fill, 0)
    for j in range(HIST // 16):
        zeros_v[pl.ds(j * 16, 16)] = jnp.zeros((16,), jnp.float32)

    @pl.when(sid == 0)
    def _():
        pltpu.sync_copy(zeros_v, shared)

    load_idx.wait()
    plsc.subcore_barrier()

    pltpu.sync_copy(ones_v, shared.at[idx_v], add=True)

    plsc.subcore_barrier()

    @pl.when(sid == 0)
    def _():
        pltpu.sync_copy(shared, out_ref.at[cid])


@functools.cache
def _sc_hist():
    return pl.kernel(
        _sc_hist_body,
        out_type=jax.ShapeDtypeStruct((NUM_CORES, HIST), jnp.float32),
        mesh=plsc.VectorSubcoreMesh(core_axis_name="c", subcore_axis_name="s"),
        scratch_types=[
            pltpu.VMEM((CHUNK,), jnp.int32),
            pltpu.VMEM((CHUNK,), jnp.float32),
            pltpu.VMEM((HIST,), jnp.float32),
            pltpu.VMEM_SHARED((HIST,), jnp.float32),
            pltpu.SemaphoreType.DMA,
        ],
    )


def _tc_scale_body(deg_ref, x_ref, o_ref):
    i = pl.program_id(0)
    deg = deg_ref[0:1, :] + deg_ref[1:2, :]
    inv = jnp.where(deg > 0.0, lax.rsqrt(deg), 0.0)
    row_ix = lax.broadcasted_iota(jnp.int32, (HIST, HIST), 0)
    col_ix = lax.broadcasted_iota(jnp.int32, (HIST, HIST), 1)
    tri = (row_ix < col_ix).astype(jnp.float32)
    s_excl = jnp.dot(
        deg, tri, precision=lax.Precision.HIGHEST, preferred_element_type=jnp.float32
    )
    s_next = s_excl + deg
    gr = (lax.broadcasted_iota(jnp.int32, (BLOCK_ROWS, 1), 0) + i * BLOCK_ROWS).astype(
        jnp.float32
    )
    member = jnp.logical_and(gr >= s_excl, gr < s_next).astype(jnp.float32)
    scale = jnp.dot(
        member, inv.reshape(HIST, 1), preferred_element_type=jnp.float32
    )
    o_ref[...] = x_ref[...] * scale


def kernel(x, batch):
    batch = batch.astype(jnp.int32)
    pad = jnp.full((PAD_N - NUM_NODES,), PAD_VALUE, jnp.int32)
    batch1d = jnp.concatenate([batch, pad])
    deg2 = _sc_hist()(batch1d)

    return pl.pallas_call(
        _tc_scale_body,
        grid=(GRID,),
        in_specs=[
            pl.BlockSpec((NUM_CORES, HIST), lambda i: (0, 0)),
            pl.BlockSpec((BLOCK_ROWS, FEAT), lambda i: (i, 0)),
        ],
        out_specs=pl.BlockSpec((BLOCK_ROWS, FEAT), lambda i: (i, 0)),
        out_shape=jax.ShapeDtypeStruct((NUM_NODES, FEAT), jnp.float32),
        compiler_params=pltpu.CompilerParams(
            dimension_semantics=("arbitrary",),
        ),
    )(deg2, x)


# speedup vs baseline: 1.3384x; 1.0022x over previous
import functools

import jax
import jax.numpy as jnp
from jax import lax
from jax.experimental import pallas as pl
from jax.experimental.pallas import tpu as pltpu
from jax.experimental.pallas import tpu_sc as plsc

NUM_NODES = 100000
FEAT = 128
NUM_GRAPHS = 64

NUM_CORES = 2
NUM_SUBCORES = 16
NUM_WORKERS = NUM_CORES * NUM_SUBCORES
CHUNK = 3200
PAD_N = NUM_WORKERS * CHUNK
PAD_VALUE = NUM_GRAPHS
HIST = 128

BLOCK_ROWS = 20000
GRID = NUM_NODES // BLOCK_ROWS


def _sc_hist_body(batch_ref, out_ref, idx_v, ones_v, zeros_v, shared, sem):
    cid = lax.axis_index("c")
    sid = lax.axis_index("s")
    wid = sid * NUM_CORES + cid

    load_idx = pltpu.async_copy(batch_ref.at[pl.ds(wid * CHUNK, CHUNK)], idx_v, sem)

    ones16 = jnp.ones((16,), jnp.float32)

    def _fill(j, carry):
        ones_v[pl.ds(j * 16, 16)] = ones16
        return carry

    lax.fori_loop(0, CHUNK // 16, _fill, 0)
    for j in range(HIST // 16):
        zeros_v[pl.ds(j * 16, 16)] = jnp.zeros((16,), jnp.float32)

    @pl.when(sid == 0)
    def _():
        pltpu.sync_copy(zeros_v, shared)

    load_idx.wait()
    plsc.subcore_barrier()

    pltpu.sync_copy(ones_v, shared.at[idx_v], add=True)

    plsc.subcore_barrier()

    @pl.when(sid == 0)
    def _():
        pltpu.sync_copy(shared, out_ref.at[cid])


@functools.cache
def _sc_hist():
    return pl.kernel(
        _sc_hist_body,
        out_type=jax.ShapeDtypeStruct((NUM_CORES, HIST), jnp.float32),
        mesh=plsc.VectorSubcoreMesh(core_axis_name="c", subcore_axis_name="s"),
        scratch_types=[
            pltpu.VMEM((CHUNK,), jnp.int32),
            pltpu.VMEM((CHUNK,), jnp.float32),
            pltpu.VMEM((HIST,), jnp.float32),
            pltpu.VMEM_SHARED((HIST,), jnp.float32),
            pltpu.SemaphoreType.DMA,
        ],
    )


def _tc_scale_body(deg_ref, x_ref, o_ref):
    i = pl.program_id(0)
    deg = deg_ref[0:1, :] + deg_ref[1:2, :]
    inv = jnp.where(deg > 0.0, lax.rsqrt(deg), 0.0)
    row_ix = lax.broadcasted_iota(jnp.int32, (HIST, HIST), 0)
    col_ix = lax.broadcasted_iota(jnp.int32, (HIST, HIST), 1)
    tri = (row_ix < col_ix).astype(jnp.float32)
    s_excl = jnp.dot(
        deg, tri, precision=lax.Precision.HIGHEST, preferred_element_type=jnp.float32
    )
    s_next = s_excl + deg
    gr = (lax.broadcasted_iota(jnp.int32, (BLOCK_ROWS, 1), 0) + i * BLOCK_ROWS).astype(
        jnp.float32
    )
    member = jnp.logical_and(gr >= s_excl, gr < s_next).astype(jnp.float32)
    scale = jnp.dot(
        member, inv.reshape(HIST, 1), preferred_element_type=jnp.float32
    )
    o_ref[...] = x_ref[...] * scale


def kernel(x, batch):
    batch = batch.astype(jnp.int32)
    pad = jnp.full((PAD_N - NUM_NODES,), PAD_VALUE, jnp.int32)
    batch1d = jnp.concatenate([batch, pad])
    deg2 = _sc_hist()(batch1d)

    return pl.pallas_call(
        _tc_scale_body,
        grid=(GRID,),
        in_specs=[
            pl.BlockSpec((NUM_CORES, HIST), lambda i: (0, 0)),
            pl.BlockSpec((BLOCK_ROWS, FEAT), lambda i: (i, 0)),
        ],
        out_specs=pl.BlockSpec((BLOCK_ROWS, FEAT), lambda i: (i, 0)),
        out_shape=jax.ShapeDtypeStruct((NUM_NODES, FEAT), jnp.float32),
        compiler_params=pltpu.CompilerParams(
            dimension_semantics=("arbitrary",),
        ),
    )(deg2, x)


# speedup vs baseline: 1.3423x; 1.0029x over previous
import functools

import jax
import jax.numpy as jnp
from jax import lax
from jax.experimental import pallas as pl
from jax.experimental.pallas import tpu as pltpu
from jax.experimental.pallas import tpu_sc as plsc

NUM_NODES = 100000
FEAT = 128
NUM_GRAPHS = 64

NUM_CORES = 2
NUM_SUBCORES = 16
NUM_WORKERS = NUM_CORES * NUM_SUBCORES
CHUNK = 3200
TAIL = NUM_NODES - (NUM_WORKERS - 1) * CHUNK
HIST = 128

BLOCK_ROWS = 20000
GRID = NUM_NODES // BLOCK_ROWS


def _sc_hist_body(batch_ref, out_ref, head_v, rest_v, ones_v, zeros_v, shared, sem):
    cid = lax.axis_index("c")
    sid = lax.axis_index("s")
    wid = sid * NUM_CORES + cid
    base = wid * CHUNK

    load_head = pltpu.async_copy(batch_ref.at[pl.ds(base, TAIL)], head_v, sem)
    is_full = wid < NUM_WORKERS - 1

    @pl.when(is_full)
    def _():
        pltpu.async_copy(batch_ref.at[pl.ds(base + TAIL, CHUNK - TAIL)], rest_v, sem)

    ones16 = jnp.ones((16,), jnp.float32)

    def _fill(j, carry):
        ones_v[pl.ds(j * 16, 16)] = ones16
        return carry

    lax.fori_loop(0, CHUNK // 16, _fill, 0)
    for j in range(HIST // 16):
        zeros_v[pl.ds(j * 16, 16)] = jnp.zeros((16,), jnp.float32)

    @pl.when(sid == 0)
    def _():
        pltpu.sync_copy(zeros_v, shared)

    load_head.wait()

    @pl.when(is_full)
    def _():
        pltpu.make_async_copy(
            batch_ref.at[pl.ds(base + TAIL, CHUNK - TAIL)], rest_v, sem
        ).wait()

    plsc.subcore_barrier()

    pltpu.sync_copy(ones_v.at[pl.ds(0, TAIL)], shared.at[head_v], add=True)

    @pl.when(is_full)
    def _():
        pltpu.sync_copy(ones_v.at[pl.ds(0, CHUNK - TAIL)], shared.at[rest_v], add=True)

    plsc.subcore_barrier()

    @pl.when(sid == 0)
    def _():
        pltpu.sync_copy(shared, out_ref.at[cid])


@functools.cache
def _sc_hist():
    return pl.kernel(
        _sc_hist_body,
        out_type=jax.ShapeDtypeStruct((NUM_CORES, HIST), jnp.float32),
        mesh=plsc.VectorSubcoreMesh(core_axis_name="c", subcore_axis_name="s"),
        scratch_types=[
            pltpu.VMEM((TAIL,), jnp.int32),
            pltpu.VMEM((CHUNK - TAIL,), jnp.int32),
            pltpu.VMEM((CHUNK,), jnp.float32),
            pltpu.VMEM((HIST,), jnp.float32),
            pltpu.VMEM_SHARED((HIST,), jnp.float32),
            pltpu.SemaphoreType.DMA,
        ],
    )


def _tc_scale_body(deg_ref, x_ref, o_ref):
    i = pl.program_id(0)
    deg = deg_ref[0:1, :] + deg_ref[1:2, :]
    inv = jnp.where(deg > 0.0, lax.rsqrt(deg), 0.0)
    row_ix = lax.broadcasted_iota(jnp.int32, (HIST, HIST), 0)
    col_ix = lax.broadcasted_iota(jnp.int32, (HIST, HIST), 1)
    tri = (row_ix < col_ix).astype(jnp.float32)
    s_excl = jnp.dot(
        deg, tri, precision=lax.Precision.HIGHEST, preferred_element_type=jnp.float32
    )
    s_next = s_excl + deg
    gr = (lax.broadcasted_iota(jnp.int32, (BLOCK_ROWS, 1), 0) + i * BLOCK_ROWS).astype(
        jnp.float32
    )
    member = jnp.logical_and(gr >= s_excl, gr < s_next).astype(jnp.float32)
    scale = jnp.dot(
        member, inv.reshape(HIST, 1), preferred_element_type=jnp.float32
    )
    o_ref[...] = x_ref[...] * scale


def kernel(x, batch):
    deg2 = _sc_hist()(batch.astype(jnp.int32))

    return pl.pallas_call(
        _tc_scale_body,
        grid=(GRID,),
        in_specs=[
            pl.BlockSpec((NUM_CORES, HIST), lambda i: (0, 0)),
            pl.BlockSpec((BLOCK_ROWS, FEAT), lambda i: (i, 0)),
        ],
        out_specs=pl.BlockSpec((BLOCK_ROWS, FEAT), lambda i: (i, 0)),
        out_shape=jax.ShapeDtypeStruct((NUM_NODES, FEAT), jnp.float32),
        compiler_params=pltpu.CompilerParams(
            dimension_semantics=("arbitrary",),
        ),
    )(deg2, x)


# speedup vs baseline: 1.3663x; 1.0178x over previous
import functools

import jax
import jax.numpy as jnp
from jax import lax
from jax.experimental import pallas as pl
from jax.experimental.pallas import tpu as pltpu
from jax.experimental.pallas import tpu_sc as plsc

NUM_NODES = 100000
FEAT = 128
NUM_GRAPHS = 64

NUM_CORES = 2
NUM_SUBCORES = 16
NUM_WORKERS = NUM_CORES * NUM_SUBCORES
CHUNK = 3200
TAIL = NUM_NODES - (NUM_WORKERS - 1) * CHUNK
HIST = 128

BLOCK_ROWS = 25000
GRID = NUM_NODES // BLOCK_ROWS


def _sc_hist_body(batch_ref, out_ref, head_v, rest_v, ones_v, zeros_v, shared, sem):
    cid = lax.axis_index("c")
    sid = lax.axis_index("s")
    wid = sid * NUM_CORES + cid
    base = wid * CHUNK

    load_head = pltpu.async_copy(batch_ref.at[pl.ds(base, TAIL)], head_v, sem)
    is_full = wid < NUM_WORKERS - 1

    @pl.when(is_full)
    def _():
        pltpu.async_copy(batch_ref.at[pl.ds(base + TAIL, CHUNK - TAIL)], rest_v, sem)

    ones16 = jnp.ones((16,), jnp.float32)

    def _fill(j, carry):
        ones_v[pl.ds(j * 16, 16)] = ones16
        return carry

    lax.fori_loop(0, CHUNK // 16, _fill, 0)
    for j in range(HIST // 16):
        zeros_v[pl.ds(j * 16, 16)] = jnp.zeros((16,), jnp.float32)

    @pl.when(sid == 0)
    def _():
        pltpu.sync_copy(zeros_v, shared)

    load_head.wait()

    @pl.when(is_full)
    def _():
        pltpu.make_async_copy(
            batch_ref.at[pl.ds(base + TAIL, CHUNK - TAIL)], rest_v, sem
        ).wait()

    plsc.subcore_barrier()

    pltpu.sync_copy(ones_v.at[pl.ds(0, TAIL)], shared.at[head_v], add=True)

    @pl.when(is_full)
    def _():
        pltpu.sync_copy(ones_v.at[pl.ds(0, CHUNK - TAIL)], shared.at[rest_v], add=True)

    plsc.subcore_barrier()

    @pl.when(sid == 0)
    def _():
        pltpu.sync_copy(shared, out_ref.at[cid])


@functools.cache
def _sc_hist():
    return pl.kernel(
        _sc_hist_body,
        out_type=jax.ShapeDtypeStruct((NUM_CORES, HIST), jnp.float32),
        mesh=plsc.VectorSubcoreMesh(core_axis_name="c", subcore_axis_name="s"),
        scratch_types=[
            pltpu.VMEM((TAIL,), jnp.int32),
            pltpu.VMEM((CHUNK - TAIL,), jnp.int32),
            pltpu.VMEM((CHUNK,), jnp.float32),
            pltpu.VMEM((HIST,), jnp.float32),
            pltpu.VMEM_SHARED((HIST,), jnp.float32),
            pltpu.SemaphoreType.DMA,
        ],
    )


def _tc_scale_body(deg_ref, x_ref, o_ref):
    i = pl.program_id(0)
    deg = deg_ref[0:1, :] + deg_ref[1:2, :]
    inv = jnp.where(deg > 0.0, lax.rsqrt(deg), 0.0)
    row_ix = lax.broadcasted_iota(jnp.int32, (HIST, HIST), 0)
    col_ix = lax.broadcasted_iota(jnp.int32, (HIST, HIST), 1)
    tri = (row_ix < col_ix).astype(jnp.float32)
    s_excl = jnp.dot(
        deg, tri, precision=lax.Precision.HIGHEST, preferred_element_type=jnp.float32
    )
    s_next = s_excl + deg
    gr = (lax.broadcasted_iota(jnp.int32, (BLOCK_ROWS, 1), 0) + i * BLOCK_ROWS).astype(
        jnp.float32
    )
    member = jnp.logical_and(gr >= s_excl, gr < s_next).astype(jnp.float32)
    scale = jnp.dot(
        member, inv.reshape(HIST, 1), preferred_element_type=jnp.float32
    )
    o_ref[...] = x_ref[...] * scale


def kernel(x, batch):
    deg2 = _sc_hist()(batch.astype(jnp.int32))

    return pl.pallas_call(
        _tc_scale_body,
        grid=(GRID,),
        in_specs=[
            pl.BlockSpec((NUM_CORES, HIST), lambda i: (0, 0)),
            pl.BlockSpec((BLOCK_ROWS, FEAT), lambda i: (i, 0)),
        ],
        out_specs=pl.BlockSpec((BLOCK_ROWS, FEAT), lambda i: (i, 0)),
        out_shape=jax.ShapeDtypeStruct((NUM_NODES, FEAT), jnp.float32),
        compiler_params=pltpu.CompilerParams(
            dimension_semantics=("arbitrary",),
            vmem_limit_bytes=110 * 1024 * 1024,
        ),
    )(deg2, x)
